# baseline (speedup 1.0000x reference)
import functools

import jax
import jax.numpy as jnp
import numpy as np
from jax import lax
from jax.experimental import pallas as pl
from jax.experimental.pallas import tpu as pltpu
from jax.experimental.pallas import tpu_sc as plsc

D = 128
K = 16
Q = 16384
GW = K * D
OUT_W = GW + 3 * K
NC, NS = 2, 16
NW = NC * NS
QPW = Q // NW
QB = 128
NQB = QPW // QB
NBLK = NQB * K
R = QPW * K // 128
QR = Q * K // 128
NBUF = 2

_LANE = np.arange(16, dtype=np.int32)
_CONSTS = np.stack(
    [_LANE, _LANE // 8]
    + [(_LANE % 8) * 16 + j for j in range(K)]
    + [np.full(16, r, dtype=np.int32) for r in range(128)]
)
_CONST_ROWS = 24
_CONST_BLOCK = np.zeros((_CONST_ROWS * 8, 16), dtype=np.int32)
_CONST_BLOCK[: _CONSTS.shape[0]] = _CONSTS
_CONST_BLOCK = _CONST_BLOCK.reshape(_CONST_ROWS, 128)


def _build_kernel():
  mesh = plsc.VectorSubcoreMesh(
      core_axis_name="c", subcore_axis_name="s", num_cores=NC, num_subcores=NS
  )

  @functools.partial(
      pl.kernel,
      out_type=jax.ShapeDtypeStruct((OUT_W, Q), jnp.float32),
      mesh=mesh,
      compiler_params=pltpu.CompilerParams(needs_layout_passes=False),
      scratch_types=[
          pltpu.VMEM((R, 128), jnp.float32),
          pltpu.VMEM((R, 128), jnp.float32),
          pltpu.VMEM((R, 128), jnp.float32),
          pltpu.VMEM((R, 128), jnp.float32),
          pltpu.VMEM((_CONST_ROWS, 128), jnp.float32),
          pltpu.VMEM((K, QPW), jnp.int32),
          [pltpu.VMEM((QB, D), jnp.float32) for _ in range(NBUF)],
          [pltpu.VMEM((D, QB), jnp.float32) for _ in range(NBUF)],
          pltpu.VMEM((3 * K, QB), jnp.float32),
          [pltpu.SemaphoreType.DMA for _ in range(NBUF)],
          [pltpu.SemaphoreType.DMA for _ in range(NBUF)],
      ],
  )
  def run(x_hbm, cat_hbm, out_hbm, idx_sl, fd_sl, fs_sl, fc_sl, const_v,
          idx_nm, rows_v, st_v, ft_v, sem_g, sem_w):
    wid = lax.axis_index("s") * NC + lax.axis_index("c")
    qbase = wid * QPW
    rbase = wid * R

    pltpu.sync_copy(cat_hbm.at[pl.ds(rbase, R), :], idx_sl)
    pltpu.sync_copy(cat_hbm.at[pl.ds(QR + rbase, R), :], fd_sl)
    pltpu.sync_copy(cat_hbm.at[pl.ds(2 * QR + rbase, R), :], fs_sl)
    pltpu.sync_copy(cat_hbm.at[pl.ds(3 * QR + rbase, R), :], fc_sl)
    pltpu.sync_copy(cat_hbm.at[pl.ds(4 * QR, _CONST_ROWS), :], const_v)

    def cv(n):
      return plsc.bitcast(
          const_v[n // 8, pl.ds((n % 8) * 16, 16)], jnp.int32
      )

    @pl.loop(0, QPW // 16)
    def _(qb):
      rowv = cv(1) + 2 * qb
      for j in range(K):
        v = plsc.load_gather(idx_sl, [rowv, cv(2 + j)])
        idx_nm[j, pl.ds(qb * 16, 16)] = plsc.bitcast(v, jnp.int32)

    def gather_cp(it, b):
      qc = it // K
      j = lax.rem(it, K)
      return pltpu.make_async_copy(
          x_hbm.at[idx_nm.at[j, pl.ds(qc * QB, QB)]], rows_v[b], sem_g[b]
      )

    def write_cp(it, b):
      qc = it // K
      j = lax.rem(it, K)
      return pltpu.make_async_copy(
          st_v[b],
          out_hbm.at[pl.ds(j * D, D), pl.ds(qbase + qc * QB, QB)],
          sem_w[b],
      )

    gather_cp(0, 0).start()

    @pl.loop(0, NBLK // NBUF)
    def _(g):
      for b in range(NBUF):
        it = g * NBUF + b
        @pl.when(g > 0)
        def _():
          write_cp(it - NBUF, b).wait()

        gather_cp(it, b).wait()

        @pl.when(it + 1 < NBLK)
        def _():
          gather_cp(it + 1, 1 - b).start()

        @pl.loop(0, D // 16)
        def _(cb):
          rowsv = cv(0) + cb * 16
          for t in range(QB):
            v = rows_v[b][t, pl.ds(cb * 16, 16)]
            plsc.store_scatter(st_v[b], [rowsv, cv(18 + t)], v)

        write_cp(it, b).start()

    for b in range(NBUF):
      write_cp(NBLK - NBUF + b, b).wait()

    for qc in range(NQB):
      @pl.loop(0, QB // 16)
      def _(tb):
        rowv = cv(1) + (16 * qc + 2 * tb)
        for sec, slab in enumerate((fd_sl, fs_sl, fc_sl)):
          for k in range(K):
            v = plsc.load_gather(slab, [rowv, cv(2 + k)])
            ft_v[sec * K + k, pl.ds(tb * 16, 16)] = v
      pltpu.sync_copy(
          ft_v, out_hbm.at[pl.ds(GW, 3 * K), pl.ds(qbase + qc * QB, QB)]
      )

  return run


def kernel(x, idx, dis, angle_t_sin, angle_t_cos):
  idx_f = jax.lax.bitcast_convert_type(idx.astype(jnp.int32), jnp.float32)
  const_f = jax.lax.bitcast_convert_type(jnp.asarray(_CONST_BLOCK), jnp.float32)
  cat = jnp.concatenate(
      [
          idx_f.reshape(QR, 128),
          dis.reshape(QR, 128),
          angle_t_sin.reshape(QR, 128),
          angle_t_cos.reshape(QR, 128),
          const_f,
      ],
      axis=0,
  )
  run = _build_kernel()
  return run(x, cat).T


# speedup vs baseline: 2.9813x; 2.9813x over previous
import functools

import jax
import jax.numpy as jnp
import numpy as np
from jax import lax
from jax.experimental import pallas as pl
from jax.experimental.pallas import tpu as pltpu
from jax.experimental.pallas import tpu_sc as plsc

D = 128
K = 16
Q = 16384
GW = K * D
OUT_W = GW + 3 * K
NC, NS = 2, 16
NW = NC * NS
QPW = Q // NW
QB = 128
NQB = QPW // QB
NBLK = NQB * K
R = QPW * K // 128
QR = Q * K // 128
NBUF = 2

_LANE = np.arange(16, dtype=np.int32)
_CONSTS = np.stack(
    [_LANE, _LANE // 8]
    + [(_LANE % 8) * 16 + j for j in range(K)]
    + [(_LANE + s) % 16 for s in range(16)]
)
_CONST_ROWS = 8
_CONST_BLOCK = np.zeros((_CONST_ROWS * 8, 16), dtype=np.int32)
_CONST_BLOCK[: _CONSTS.shape[0]] = _CONSTS
_CONST_BLOCK = _CONST_BLOCK.reshape(_CONST_ROWS, 128)


def _build_kernel():
  mesh = plsc.VectorSubcoreMesh(
      core_axis_name="c", subcore_axis_name="s", num_cores=NC, num_subcores=NS
  )

  @functools.partial(
      pl.kernel,
      out_type=jax.ShapeDtypeStruct((OUT_W, Q), jnp.float32),
      mesh=mesh,
      compiler_params=pltpu.CompilerParams(needs_layout_passes=False),
      scratch_types=[
          pltpu.VMEM((R, 128), jnp.float32),
          pltpu.VMEM((R, 128), jnp.float32),
          pltpu.VMEM((R, 128), jnp.float32),
          pltpu.VMEM((R, 128), jnp.float32),
          pltpu.VMEM((_CONST_ROWS, 128), jnp.float32),
          pltpu.VMEM((K, QPW), jnp.int32),
          [pltpu.VMEM((QB, D), jnp.float32) for _ in range(NBUF)],
          [pltpu.VMEM((D, QB), jnp.float32) for _ in range(NBUF)],
          pltpu.VMEM((3 * K, QB), jnp.float32),
          [pltpu.SemaphoreType.DMA for _ in range(NBUF)],
          [pltpu.SemaphoreType.DMA for _ in range(NBUF)],
      ],
  )
  def run(x_hbm, cat_hbm, out_hbm, idx_sl, fd_sl, fs_sl, fc_sl, const_v,
          idx_nm, rows_v, st_v, ft_v, sem_g, sem_w):
    wid = lax.axis_index("s") * NC + lax.axis_index("c")
    qbase = wid * QPW
    rbase = wid * R

    pltpu.sync_copy(cat_hbm.at[pl.ds(rbase, R), :], idx_sl)
    pltpu.sync_copy(cat_hbm.at[pl.ds(QR + rbase, R), :], fd_sl)
    pltpu.sync_copy(cat_hbm.at[pl.ds(2 * QR + rbase, R), :], fs_sl)
    pltpu.sync_copy(cat_hbm.at[pl.ds(3 * QR + rbase, R), :], fc_sl)
    pltpu.sync_copy(cat_hbm.at[pl.ds(4 * QR, _CONST_ROWS), :], const_v)

    def cv(n):
      return plsc.bitcast(
          const_v[n // 8, pl.ds((n % 8) * 16, 16)], jnp.int32
      )

    @pl.loop(0, QPW // 16)
    def _(qb):
      rowv = cv(1) + 2 * qb
      for j in range(K):
        v = plsc.load_gather(idx_sl, [rowv, cv(2 + j)])
        idx_nm[j, pl.ds(qb * 16, 16)] = plsc.bitcast(v, jnp.int32)

    def gather_cp(it, b):
      qc = it // K
      j = lax.rem(it, K)
      return pltpu.make_async_copy(
          x_hbm.at[idx_nm.at[j, pl.ds(qc * QB, QB)]], rows_v[b], sem_g[b]
      )

    def write_cp(it, b):
      qc = it // K
      j = lax.rem(it, K)
      return pltpu.make_async_copy(
          st_v[b],
          out_hbm.at[pl.ds(j * D, D), pl.ds(qbase + qc * QB, QB)],
          sem_w[b],
      )

    gather_cp(0, 0).start()

    @pl.loop(0, NBLK // NBUF)
    def _(g):
      for b in range(NBUF):
        it = g * NBUF + b
        @pl.when(g > 0)
        def _():
          write_cp(it - NBUF, b).wait()

        gather_cp(it, b).wait()

        @pl.when(it + 1 < NBLK)
        def _():
          gather_cp(it + 1, 1 - b).start()

        @pl.loop(0, QB // 16)
        def _(tb):
          va = cv(0) + tb * 16
          diags = [cv(18 + s) for s in range(16)]
          for cb in range(D // 16):
            for s in range(16):
              vb = diags[s] + cb * 16
              v = plsc.load_gather(rows_v[b], [va, vb])
              plsc.store_scatter(st_v[b], [vb, va], v)

        write_cp(it, b).start()

    for b in range(NBUF):
      write_cp(NBLK - NBUF + b, b).wait()

    for qc in range(NQB):
      @pl.loop(0, QB // 16)
      def _(tb):
        rowv = cv(1) + (16 * qc + 2 * tb)
        for sec, slab in enumerate((fd_sl, fs_sl, fc_sl)):
          for k in range(K):
            v = plsc.load_gather(slab, [rowv, cv(2 + k)])
            ft_v[sec * K + k, pl.ds(tb * 16, 16)] = v
      pltpu.sync_copy(
          ft_v, out_hbm.at[pl.ds(GW, 3 * K), pl.ds(qbase + qc * QB, QB)]
      )

  return run


def kernel(x, idx, dis, angle_t_sin, angle_t_cos):
  idx_f = jax.lax.bitcast_convert_type(idx.astype(jnp.int32), jnp.float32)
  const_f = jax.lax.bitcast_convert_type(jnp.asarray(_CONST_BLOCK), jnp.float32)
  cat = jnp.concatenate(
      [
          idx_f.reshape(QR, 128),
          dis.reshape(QR, 128),
          angle_t_sin.reshape(QR, 128),
          angle_t_cos.reshape(QR, 128),
          const_f,
      ],
      axis=0,
  )
  run = _build_kernel()
  return run(x, cat).T


# speedup vs baseline: 3.9542x; 1.3263x over previous
import functools

import jax
import jax.numpy as jnp
import numpy as np
from jax import lax
from jax.experimental import pallas as pl
from jax.experimental.pallas import tpu as pltpu
from jax.experimental.pallas import tpu_sc as plsc

D = 128
K = 16
Q = 16384
GW = K * D
OUT_W = GW + 3 * K
NC, NS = 2, 16
NW = NC * NS
QPW = Q // NW
QB = 128
NQB = QPW // QB
NBLK = NQB * K
NBUF = 2

_LANE = np.arange(16, dtype=np.int32)
_CONSTS = np.stack([_LANE] + [(_LANE + s) % 16 for s in range(16)])
_CONST_BLOCK = np.zeros((8, 128), dtype=np.int32)
_CONST_BLOCK.reshape(-1, 16)[: _CONSTS.shape[0]] = _CONSTS


def _build_kernel():
  mesh = plsc.VectorSubcoreMesh(
      core_axis_name="c", subcore_axis_name="s", num_cores=NC, num_subcores=NS
  )

  @functools.partial(
      pl.kernel,
      out_type=jax.ShapeDtypeStruct((OUT_W, Q), jnp.float32),
      mesh=mesh,
      compiler_params=pltpu.CompilerParams(needs_layout_passes=False),
      scratch_types=[
          pltpu.VMEM((8, 128), jnp.int32),
          pltpu.VMEM((K, QPW), jnp.int32),
          [pltpu.VMEM((QB, D), jnp.float32) for _ in range(NBUF)],
          [pltpu.VMEM((D, QB), jnp.float32) for _ in range(NBUF)],
          [pltpu.SemaphoreType.DMA for _ in range(NBUF)],
          [pltpu.SemaphoreType.DMA for _ in range(NBUF)],
          pltpu.SemaphoreType.DMA,
      ],
  )
  def run(x_hbm, idxt_hbm, dist_hbm, sint_hbm, cost_hbm, const_hbm, out_hbm,
          const_v, idx_nm, rows_v, st_v, sem_g, sem_w, sem_f):
    wid = lax.axis_index("s") * NC + lax.axis_index("c")
    qbase = wid * QPW

    ft_cps = [
        pltpu.make_async_copy(
            src.at[:, pl.ds(qbase, QPW)],
            out_hbm.at[pl.ds(GW + sec * K, K), pl.ds(qbase, QPW)],
            sem_f,
        )
        for sec, src in enumerate((dist_hbm, sint_hbm, cost_hbm))
    ]
    for cp in ft_cps:
      cp.start()

    pltpu.sync_copy(const_hbm, const_v)
    pltpu.sync_copy(idxt_hbm.at[:, pl.ds(qbase, QPW)], idx_nm)

    def cv(n):
      return const_v[n // 8, pl.ds((n % 8) * 16, 16)]

    def gather_cp(it, b):
      qc = it // K
      j = lax.rem(it, K)
      return pltpu.make_async_copy(
          x_hbm.at[idx_nm.at[j, pl.ds(qc * QB, QB)]], rows_v[b], sem_g[b]
      )

    def write_cp(it, b):
      qc = it // K
      j = lax.rem(it, K)
      return pltpu.make_async_copy(
          st_v[b],
          out_hbm.at[pl.ds(j * D, D), pl.ds(qbase + qc * QB, QB)],
          sem_w[b],
      )

    gather_cp(0, 0).start()

    @pl.loop(0, NBLK // NBUF)
    def _(g):
      for b in range(NBUF):
        it = g * NBUF + b
        @pl.when(g > 0)
        def _():
          write_cp(it - NBUF, b).wait()

        gather_cp(it, b).wait()

        @pl.when(it + 1 < NBLK)
        def _():
          gather_cp(it + 1, 1 - b).start()

        @pl.loop(0, QB // 16)
        def _(tb):
          va = cv(0) + tb * 16
          diags = [cv(1 + s) for s in range(16)]
          for cb in range(D // 16):
            for s in range(16):
              vb = diags[s] + cb * 16
              v = plsc.load_gather(rows_v[b], [va, vb])
              plsc.store_scatter(st_v[b], [vb, va], v)

        write_cp(it, b).start()

    for b in range(NBUF):
      write_cp(NBLK - NBUF + b, b).wait()
    for cp in ft_cps:
      cp.wait()

  return run


def kernel(x, idx, dis, angle_t_sin, angle_t_cos):
  run = _build_kernel()
  out_t = run(
      x,
      idx.astype(jnp.int32).T,
      dis.T,
      angle_t_sin.T,
      angle_t_cos.T,
      jnp.asarray(_CONST_BLOCK),
  )
  return out_t.T


# speedup vs baseline: 5.0145x; 1.2681x over previous
import functools

import jax
import jax.numpy as jnp
import numpy as np
from jax import lax
from jax.experimental import pallas as pl
from jax.experimental.pallas import tpu as pltpu
from jax.experimental.pallas import tpu_sc as plsc

D = 128
K = 16
Q = 16384
GW = K * D
OUT_W = GW + 3 * K
NC, NS = 2, 16
NW = NC * NS
QPW = Q // NW
QB = 128
NQB = QPW // QB
NBLK = NQB * K
NBUF = 2

_LANE = np.arange(16, dtype=np.int32)
_CONSTS = np.stack([_LANE] + [(_LANE + s) % 16 for s in range(16)])
_CONST_BLOCK = np.zeros((8, 128), dtype=np.int32)
_CONST_BLOCK.reshape(-1, 16)[: _CONSTS.shape[0]] = _CONSTS


def _build_kernel():
  mesh = plsc.VectorSubcoreMesh(
      core_axis_name="c", subcore_axis_name="s", num_cores=NC, num_subcores=NS
  )

  @functools.partial(
      pl.kernel,
      out_type=jax.ShapeDtypeStruct((OUT_W, Q), jnp.float32),
      mesh=mesh,
      compiler_params=pltpu.CompilerParams(needs_layout_passes=False),
      scratch_types=[
          pltpu.VMEM((8, 128), jnp.int32),
          pltpu.VMEM((K, QPW), jnp.int32),
          [pltpu.VMEM((QB, D), jnp.float32) for _ in range(NBUF)],
          [pltpu.VMEM((D, QB), jnp.float32) for _ in range(NBUF)],
          [pltpu.SemaphoreType.DMA for _ in range(NBUF)],
          [pltpu.SemaphoreType.DMA for _ in range(NBUF)],
          pltpu.SemaphoreType.DMA,
      ],
  )
  def run(x_hbm, idxt_hbm, dist_hbm, sint_hbm, cost_hbm, const_hbm, out_hbm,
          const_v, idx_nm, rows_v, st_v, sem_g, sem_w, sem_f):
    wid = lax.axis_index("s") * NC + lax.axis_index("c")
    qbase = wid * QPW

    ft_cps = [
        pltpu.make_async_copy(
            src.at[:, pl.ds(qbase, QPW)],
            out_hbm.at[pl.ds(GW + sec * K, K), pl.ds(qbase, QPW)],
            sem_f,
        )
        for sec, src in enumerate((dist_hbm, sint_hbm, cost_hbm))
    ]
    for cp in ft_cps:
      cp.start()

    pltpu.sync_copy(const_hbm, const_v)
    pltpu.sync_copy(idxt_hbm.at[:, pl.ds(qbase, QPW)], idx_nm)

    def cv(n):
      return const_v[n // 8, pl.ds((n % 8) * 16, 16)]

    def gather_cp(it, b):
      qc = it // K
      j = lax.rem(it, K)
      return pltpu.make_async_copy(
          x_hbm.at[idx_nm.at[j, pl.ds(qc * QB, QB)]], rows_v[b], sem_g[b]
      )

    def write_cp(it, b):
      qc = it // K
      j = lax.rem(it, K)
      return pltpu.make_async_copy(
          st_v[b],
          out_hbm.at[pl.ds(j * D, D), pl.ds(qbase + qc * QB, QB)],
          sem_w[b],
      )

    gather_cp(0, 0).start()

    @pl.loop(0, NBLK // NBUF)
    def _(g):
      for b in range(NBUF):
        it = g * NBUF + b
        @pl.when(g > 0)
        def _():
          write_cp(it - NBUF, b).wait()

        gather_cp(it, b).wait()

        @pl.when(it + 1 < NBLK)
        def _():
          gather_cp(it + 1, 1 - b).start()

        diags = [cv(1 + s) for s in range(16)]

        @plsc.parallel_loop(0, QB // 16)
        def _(tb):
          va = cv(0) + tb * 16
          for cb in range(D // 16):
            vs = [
                plsc.load_gather(rows_v[b], [va, diags[s] + cb * 16])
                for s in range(16)
            ]
            for s in range(16):
              plsc.store_scatter(st_v[b], [diags[s] + cb * 16, va], vs[s])

        write_cp(it, b).start()

    for b in range(NBUF):
      write_cp(NBLK - NBUF + b, b).wait()
    for cp in ft_cps:
      cp.wait()

  return run


def kernel(x, idx, dis, angle_t_sin, angle_t_cos):
  run = _build_kernel()
  out_t = run(
      x,
      idx.astype(jnp.int32).T,
      dis.T,
      angle_t_sin.T,
      angle_t_cos.T,
      jnp.asarray(_CONST_BLOCK),
  )
  return out_t.T
